# baseline (speedup 1.0000x reference)
import functools

import jax
import jax.numpy as jnp
from jax import lax
from jax.experimental import pallas as pl
from jax.experimental.pallas import tpu as pltpu
from jax.experimental.pallas import tpu_sc as plsc

N, S, D, K = 4096, 200, 64, 50
NC, NS = 2, 16
NW = NC * NS
CW = N // NW
CH = 16
NBUF = 2
_DSLICES = [(0, 24), (24, 24), (48, 16)]

_TASKS = []
for _c in range((K + CH - 1) // CH):
    _n = min(CH, K - _c * CH)
    for _do, _dl in _DSLICES:
        _TASKS.append((_c, _do, _dl, _c * CH, _n))

_mesh = plsc.VectorSubcoreMesh(core_axis_name="c", subcore_axis_name="s")


@functools.partial(
    pl.kernel,
    mesh=_mesh,
    out_type=jax.ShapeDtypeStruct((K, D, N), jnp.float32),
    scratch_types=[
        pltpu.VMEM((64,), jnp.int32),
        pltpu.VMEM((4, 16), jnp.int32),
        *[pltpu.VMEM((CH, 24, CW), jnp.float32) for _ in range(NBUF)],
        pltpu.SemaphoreType.DMA,
        pltpu.SemaphoreType.DMA,
    ],
)
def _sc_gather(in_t, ind_hbm, out_t, raw_v, idx_v, *rest):
    bufs = list(rest[:NBUF])
    gsem, wsem = rest[NBUF], rest[NBUF + 1]
    wid = lax.axis_index("s") * NC + lax.axis_index("c")
    c0 = wid * CW

    pltpu.sync_copy(ind_hbm, raw_v.at[pl.ds(0, K)])
    for c in range(4):
        v = raw_v[pl.ds(c * 16, 16)]
        idx_v[c, pl.ds(0, 16)] = lax.rem(lax.rem(v, S) + S, S)

    def gather(t):
        c, do, dl, _, n = _TASKS[t]
        pltpu.make_async_copy(
            in_t.at[idx_v.at[c, pl.ds(0, n)], pl.ds(do, dl), pl.ds(c0, CW)],
            bufs[t % NBUF].at[pl.ds(0, n), pl.ds(0, dl)],
            gsem,
        ).start()

    def wait_gather(t):
        _, _, dl, _, n = _TASKS[t]
        pltpu.make_async_copy(
            in_t.at[pl.ds(0, n), pl.ds(0, dl), pl.ds(c0, CW)],
            bufs[t % NBUF].at[pl.ds(0, n), pl.ds(0, dl)],
            gsem,
        ).wait()

    def write(t):
        _, do, dl, j0, n = _TASKS[t]
        pltpu.make_async_copy(
            bufs[t % NBUF].at[pl.ds(0, n), pl.ds(0, dl)],
            out_t.at[pl.ds(j0, n), pl.ds(do, dl), pl.ds(c0, CW)],
            wsem,
        ).start()

    def wait_write(t):
        _, _, dl, _, n = _TASKS[t]
        pltpu.make_async_copy(
            bufs[t % NBUF].at[pl.ds(0, n), pl.ds(0, dl)],
            out_t.at[pl.ds(0, n), pl.ds(0, dl), pl.ds(c0, CW)],
            wsem,
        ).wait()

    T = len(_TASKS)
    for t in range(min(NBUF - 1, T)):
        gather(t)
    waited = 0
    for t in range(T):
        k = t + NBUF - 1
        if k < T:
            if k - NBUF >= 0:
                wait_write(k - NBUF)
                waited = k - NBUF + 1
            gather(k)
        wait_gather(t)
        write(t)
    for t in range(waited, T):
        wait_write(t)


def kernel(inputs, indices):
    in_t = jnp.transpose(inputs, (1, 2, 0))
    out_t = _sc_gather(in_t, indices.astype(jnp.int32))
    return jnp.transpose(out_t, (2, 0, 1))


# speedup vs baseline: 1.0340x; 1.0340x over previous
import functools

import jax
import jax.numpy as jnp
from jax import lax
from jax.experimental import pallas as pl
from jax.experimental.pallas import tpu as pltpu
from jax.experimental.pallas import tpu_sc as plsc

N, S, D, K = 4096, 200, 64, 50
NC, NS = 2, 16
NW = NC * NS
CW = N // NW
CH = 8
DH = D // 2
NBUF = 3

_TASKS = []
for _c in range(4):
    for _h in range(2):
        _j0 = _c * 16 + _h * CH
        if _j0 >= K:
            continue
        _n = min(CH, K - _j0)
        for _dh in range(2):
            _TASKS.append((_c, _h, _dh, _j0, _n))

_mesh = plsc.VectorSubcoreMesh(core_axis_name="c", subcore_axis_name="s")


@functools.partial(
    pl.kernel,
    mesh=_mesh,
    out_type=jax.ShapeDtypeStruct((K, D, N), jnp.float32),
    scratch_types=[
        pltpu.VMEM((64,), jnp.int32),
        pltpu.VMEM((4, 16), jnp.int32),
        pltpu.VMEM((CH, DH, CW), jnp.float32),
        pltpu.VMEM((CH, DH, CW), jnp.float32),
        pltpu.VMEM((CH, DH, CW), jnp.float32),
        pltpu.SemaphoreType.DMA,
        pltpu.SemaphoreType.DMA,
    ],
)
def _sc_gather(in_t, ind_hbm, out_t, raw_v, idx_v, buf0, buf1, buf2, gsem, wsem):
    wid = lax.axis_index("s") * NC + lax.axis_index("c")
    c0 = wid * CW
    bufs = [buf0, buf1, buf2]

    pltpu.sync_copy(ind_hbm, raw_v.at[pl.ds(0, K)])
    for c in range(4):
        v = raw_v[pl.ds(c * 16, 16)]
        idx_v[c, pl.ds(0, 16)] = lax.rem(lax.rem(v, S) + S, S)

    def gather(t):
        c, h, dh, _, n = _TASKS[t]
        pltpu.make_async_copy(
            in_t.at[idx_v.at[c, pl.ds(h * CH, n)],
                    pl.ds(dh * DH, DH), pl.ds(c0, CW)],
            bufs[t % NBUF].at[pl.ds(0, n)],
            gsem,
        ).start()

    def wait_gather(t):
        _, _, _, _, n = _TASKS[t]
        pltpu.make_async_copy(
            in_t.at[pl.ds(0, n), pl.ds(0, DH), pl.ds(c0, CW)],
            bufs[t % NBUF].at[pl.ds(0, n)],
            gsem,
        ).wait()

    def write(t):
        _, _, dh, j0, n = _TASKS[t]
        pltpu.make_async_copy(
            bufs[t % NBUF].at[pl.ds(0, n)],
            out_t.at[pl.ds(j0, n), pl.ds(dh * DH, DH), pl.ds(c0, CW)],
            wsem,
        ).start()

    def wait_write(t):
        _, _, _, _, n = _TASKS[t]
        pltpu.make_async_copy(
            bufs[t % NBUF].at[pl.ds(0, n)],
            out_t.at[pl.ds(0, n), pl.ds(0, DH), pl.ds(c0, CW)],
            wsem,
        ).wait()

    T = len(_TASKS)
    for t in range(min(NBUF - 1, T)):
        gather(t)
    waited = 0
    for t in range(T):
        k = t + NBUF - 1
        if k < T:
            if k - NBUF >= 0:
                wait_write(k - NBUF)
                waited = k - NBUF + 1
            gather(k)
        wait_gather(t)
        write(t)
    for t in range(waited, T):
        wait_write(t)


def kernel(inputs, indices):
    in_t = jnp.transpose(inputs, (1, 2, 0))
    out_t = _sc_gather(in_t, indices.astype(jnp.int32))
    return jnp.transpose(out_t, (2, 0, 1))


# speedup vs baseline: 1.0360x; 1.0019x over previous
import functools

import jax
import jax.numpy as jnp
from jax import lax
from jax.experimental import pallas as pl
from jax.experimental.pallas import tpu as pltpu
from jax.experimental.pallas import tpu_sc as plsc

N, S, D, K = 4096, 200, 64, 50
NC, NS = 2, 16
NW = NC * NS
CW = N // NW
CH = 8
NBUF = 3
_DSLICES = [(0, 40), (40, 24)]
_DMAX = max(_dl for _, _dl in _DSLICES)

_TASKS = []
for _c in range(4):
    for _h in range(2):
        _j0 = _c * 16 + _h * CH
        if _j0 >= K:
            continue
        _n = min(CH, K - _j0)
        for _do, _dl in _DSLICES:
            _TASKS.append((_c, _h, _do, _dl, _j0, _n))

_mesh = plsc.VectorSubcoreMesh(core_axis_name="c", subcore_axis_name="s")


@functools.partial(
    pl.kernel,
    mesh=_mesh,
    out_type=jax.ShapeDtypeStruct((K, D, N), jnp.float32),
    scratch_types=[
        pltpu.VMEM((64,), jnp.int32),
        pltpu.VMEM((4, 16), jnp.int32),
        *[pltpu.VMEM((CH, _DMAX, CW), jnp.float32) for _ in range(NBUF)],
        pltpu.SemaphoreType.DMA,
        pltpu.SemaphoreType.DMA,
    ],
)
def _sc_gather(in_t, ind_hbm, out_t, raw_v, idx_v, *rest):
    bufs = list(rest[:NBUF])
    gsem, wsem = rest[NBUF], rest[NBUF + 1]
    wid = lax.axis_index("s") * NC + lax.axis_index("c")
    c0 = wid * CW

    pltpu.sync_copy(ind_hbm, raw_v.at[pl.ds(0, K)])
    for c in range(4):
        v = raw_v[pl.ds(c * 16, 16)]
        idx_v[c, pl.ds(0, 16)] = lax.rem(lax.rem(v, S) + S, S)

    def gather(t):
        c, h, do, dl, _, n = _TASKS[t]
        pltpu.make_async_copy(
            in_t.at[idx_v.at[c, pl.ds(h * CH, n)], pl.ds(do, dl), pl.ds(c0, CW)],
            bufs[t % NBUF].at[pl.ds(0, n), pl.ds(0, dl)],
            gsem,
        ).start()

    def wait_gather(t):
        _, _, _, dl, _, n = _TASKS[t]
        pltpu.make_async_copy(
            in_t.at[pl.ds(0, n), pl.ds(0, dl), pl.ds(c0, CW)],
            bufs[t % NBUF].at[pl.ds(0, n), pl.ds(0, dl)],
            gsem,
        ).wait()

    def write(t):
        _, _, do, dl, j0, n = _TASKS[t]
        pltpu.make_async_copy(
            bufs[t % NBUF].at[pl.ds(0, n), pl.ds(0, dl)],
            out_t.at[pl.ds(j0, n), pl.ds(do, dl), pl.ds(c0, CW)],
            wsem,
        ).start()

    def wait_write(t):
        _, _, _, dl, _, n = _TASKS[t]
        pltpu.make_async_copy(
            bufs[t % NBUF].at[pl.ds(0, n), pl.ds(0, dl)],
            out_t.at[pl.ds(0, n), pl.ds(0, dl), pl.ds(c0, CW)],
            wsem,
        ).wait()

    T = len(_TASKS)
    for t in range(min(NBUF - 1, T)):
        gather(t)
    waited = 0
    for t in range(T):
        k = t + NBUF - 1
        if k < T:
            if k - NBUF >= 0:
                wait_write(k - NBUF)
                waited = k - NBUF + 1
            gather(k)
        wait_gather(t)
        write(t)
    for t in range(waited, T):
        wait_write(t)


def kernel(inputs, indices):
    in_t = jnp.transpose(inputs, (1, 2, 0))
    out_t = _sc_gather(in_t, indices.astype(jnp.int32))
    return jnp.transpose(out_t, (2, 0, 1))


# speedup vs baseline: 1.0500x; 1.0135x over previous
import functools

import jax
import jax.numpy as jnp
from jax import lax
from jax.experimental import pallas as pl
from jax.experimental.pallas import tpu as pltpu
from jax.experimental.pallas import tpu_sc as plsc

N, S, D, K = 4096, 200, 64, 50
NC, NS = 2, 16
NW = NC * NS
NG = NW // 2
CW = N // NG
CH = 8
DH = 16
NBUF = 3

_TASKS = []
for _c in range(4):
    for _h in range(2):
        _j0 = _c * 16 + _h * CH
        if _j0 >= K:
            continue
        _n = min(CH, K - _j0)
        for _ds in range(2):
            _TASKS.append((_c, _h, _ds, _j0, _n))

_mesh = plsc.VectorSubcoreMesh(core_axis_name="c", subcore_axis_name="s")


@functools.partial(
    pl.kernel,
    mesh=_mesh,
    out_type=jax.ShapeDtypeStruct((K, D, N), jnp.float32),
    scratch_types=[
        pltpu.VMEM((64,), jnp.int32),
        pltpu.VMEM((4, 16), jnp.int32),
        *[pltpu.VMEM((CH, DH, CW), jnp.float32) for _ in range(NBUF)],
        pltpu.SemaphoreType.DMA,
        pltpu.SemaphoreType.DMA,
    ],
)
def _sc_gather(in_t, ind_hbm, out_t, raw_v, idx_v, *rest):
    bufs = list(rest[:NBUF])
    gsem, wsem = rest[NBUF], rest[NBUF + 1]
    wid = lax.axis_index("s") * NC + lax.axis_index("c")
    g = wid // 2
    u = wid - g * 2
    c0 = pl.multiple_of(g * CW, CW)
    d0 = pl.multiple_of(u * (D // 2), D // 2)

    pltpu.sync_copy(ind_hbm, raw_v.at[pl.ds(0, K)])
    for c in range(4):
        v = raw_v[pl.ds(c * 16, 16)]
        idx_v[c, pl.ds(0, 16)] = lax.rem(lax.rem(v, S) + S, S)

    def gather(t):
        c, h, ds_, _, n = _TASKS[t]
        pltpu.make_async_copy(
            in_t.at[idx_v.at[c, pl.ds(h * CH, n)],
                    pl.ds(d0 + ds_ * DH, DH), pl.ds(c0, CW)],
            bufs[t % NBUF].at[pl.ds(0, n)],
            gsem,
        ).start()

    def wait_gather(t):
        _, _, _, _, n = _TASKS[t]
        pltpu.make_async_copy(
            in_t.at[pl.ds(0, n), pl.ds(0, DH), pl.ds(c0, CW)],
            bufs[t % NBUF].at[pl.ds(0, n)],
            gsem,
        ).wait()

    def write(t):
        _, _, ds_, j0, n = _TASKS[t]
        pltpu.make_async_copy(
            bufs[t % NBUF].at[pl.ds(0, n)],
            out_t.at[pl.ds(j0, n), pl.ds(d0 + ds_ * DH, DH), pl.ds(c0, CW)],
            wsem,
        ).start()

    def wait_write(t):
        _, _, _, _, n = _TASKS[t]
        pltpu.make_async_copy(
            bufs[t % NBUF].at[pl.ds(0, n)],
            out_t.at[pl.ds(0, n), pl.ds(0, DH), pl.ds(c0, CW)],
            wsem,
        ).wait()

    T = len(_TASKS)
    for t in range(min(NBUF - 1, T)):
        gather(t)
    waited = 0
    for t in range(T):
        k = t + NBUF - 1
        if k < T:
            if k - NBUF >= 0:
                wait_write(k - NBUF)
                waited = k - NBUF + 1
            gather(k)
        wait_gather(t)
        write(t)
    for t in range(waited, T):
        wait_write(t)


def kernel(inputs, indices):
    in_t = jnp.transpose(inputs, (1, 2, 0))
    out_t = _sc_gather(in_t, indices.astype(jnp.int32))
    return jnp.transpose(out_t, (2, 0, 1))


# speedup vs baseline: 1.0575x; 1.0071x over previous
import functools

import jax
import jax.numpy as jnp
from jax import lax
from jax.experimental import pallas as pl
from jax.experimental.pallas import tpu as pltpu
from jax.experimental.pallas import tpu_sc as plsc

N, S, D, K = 4096, 200, 64, 50
NC, NS = 2, 16
NW = NC * NS
NG = NW // 4
CW = N // NG
CH = 8
DH = 8
NBUF = 3

_TASKS = []
for _c in range(4):
    for _h in range(2):
        _j0 = _c * 16 + _h * CH
        if _j0 >= K:
            continue
        _n = min(CH, K - _j0)
        for _ds in range(2):
            _TASKS.append((_c, _h, _ds, _j0, _n))

_mesh = plsc.VectorSubcoreMesh(core_axis_name="c", subcore_axis_name="s")


@functools.partial(
    pl.kernel,
    mesh=_mesh,
    out_type=jax.ShapeDtypeStruct((K, D, N), jnp.float32),
    scratch_types=[
        pltpu.VMEM((64,), jnp.int32),
        pltpu.VMEM((4, 16), jnp.int32),
        *[pltpu.VMEM((CH, DH, CW), jnp.float32) for _ in range(NBUF)],
        pltpu.SemaphoreType.DMA,
        pltpu.SemaphoreType.DMA,
    ],
)
def _sc_gather(in_t, ind_hbm, out_t, raw_v, idx_v, *rest):
    bufs = list(rest[:NBUF])
    gsem, wsem = rest[NBUF], rest[NBUF + 1]
    wid = lax.axis_index("s") * NC + lax.axis_index("c")
    g = wid // 4
    u = wid - g * 4
    c0 = pl.multiple_of(g * CW, CW)
    d0 = pl.multiple_of(u * (D // 4), D // 4)

    pltpu.sync_copy(ind_hbm, raw_v.at[pl.ds(0, K)])
    for c in range(4):
        v = raw_v[pl.ds(c * 16, 16)]
        idx_v[c, pl.ds(0, 16)] = lax.rem(lax.rem(v, S) + S, S)

    def gather(t):
        c, h, ds_, _, n = _TASKS[t]
        pltpu.make_async_copy(
            in_t.at[idx_v.at[c, pl.ds(h * CH, n)],
                    pl.ds(d0 + ds_ * DH, DH), pl.ds(c0, CW)],
            bufs[t % NBUF].at[pl.ds(0, n)],
            gsem,
        ).start()

    def wait_gather(t):
        _, _, _, _, n = _TASKS[t]
        pltpu.make_async_copy(
            in_t.at[pl.ds(0, n), pl.ds(0, DH), pl.ds(c0, CW)],
            bufs[t % NBUF].at[pl.ds(0, n)],
            gsem,
        ).wait()

    def write(t):
        _, _, ds_, j0, n = _TASKS[t]
        pltpu.make_async_copy(
            bufs[t % NBUF].at[pl.ds(0, n)],
            out_t.at[pl.ds(j0, n), pl.ds(d0 + ds_ * DH, DH), pl.ds(c0, CW)],
            wsem,
        ).start()

    def wait_write(t):
        _, _, _, _, n = _TASKS[t]
        pltpu.make_async_copy(
            bufs[t % NBUF].at[pl.ds(0, n)],
            out_t.at[pl.ds(0, n), pl.ds(0, DH), pl.ds(c0, CW)],
            wsem,
        ).wait()

    T = len(_TASKS)
    for t in range(min(NBUF - 1, T)):
        gather(t)
    waited = 0
    for t in range(T):
        k = t + NBUF - 1
        if k < T:
            if k - NBUF >= 0:
                wait_write(k - NBUF)
                waited = k - NBUF + 1
            gather(k)
        wait_gather(t)
        write(t)
    for t in range(waited, T):
        wait_write(t)


def kernel(inputs, indices):
    in_t = jnp.transpose(inputs, (1, 2, 0))
    out_t = _sc_gather(in_t, indices.astype(jnp.int32))
    return jnp.transpose(out_t, (2, 0, 1))
